# baseline (speedup 1.0000x reference)
import jax
import jax.numpy as jnp
from jax.experimental import pallas as pl
from jax.experimental.pallas import tpu as pltpu

N = 4096
D_IN = 512
D_OUT = 512
BI = 256
NI = N // BI
XH = N // 2


def _fused_kernel(x_hbm, w_hbm, adj_hbm, o_hbm,
                  x_v, w_v, h_v, a0, a1, o0, o1,
                  sx0, sx1, sw, sa0, sa1, so0, so1):
    abuf = [a0, a1]
    asem = [sa0, sa1]
    obuf = [o0, o1]
    osem = [so0, so1]
    xsem = [sx0, sx1]

    cw = pltpu.make_async_copy(w_hbm, w_v, sw)
    cw.start()
    for j in range(2):
        pltpu.make_async_copy(
            x_hbm.at[pl.ds(j * XH, XH), :], x_v.at[pl.ds(j * XH, XH), :],
            xsem[j],
        ).start()
    for j in range(2):
        pltpu.make_async_copy(
            adj_hbm.at[pl.ds(j * BI, BI), :], abuf[j], asem[j]
        ).start()

    cw.wait()
    for j in range(2):
        pltpu.make_async_copy(
            x_hbm.at[pl.ds(j * XH, XH), :], x_v.at[pl.ds(j * XH, XH), :],
            xsem[j],
        ).wait()
        h_v[pl.ds(j * XH, XH), :] = jnp.dot(
            x_v[pl.ds(j * XH, XH), :], w_v[...],
            preferred_element_type=jnp.float32,
        ).astype(jnp.bfloat16)

    for i in range(NI):
        b = i % 2
        pltpu.make_async_copy(
            adj_hbm.at[pl.ds(i * BI, BI), :], abuf[b], asem[b]
        ).wait()
        a = abuf[b][...]
        deg = jnp.sum(a, axis=1, keepdims=True)
        acc = jnp.dot(
            a.astype(jnp.bfloat16), h_v[...],
            preferred_element_type=jnp.float32,
        )
        if i >= 2:
            pltpu.make_async_copy(
                obuf[b], o_hbm.at[pl.ds((i - 2) * BI, BI), :], osem[b]
            ).wait()
        obuf[b][...] = acc / deg
        pltpu.make_async_copy(
            obuf[b], o_hbm.at[pl.ds(i * BI, BI), :], osem[b]
        ).start()
        if i + 2 < NI:
            pltpu.make_async_copy(
                adj_hbm.at[pl.ds((i + 2) * BI, BI), :], abuf[b], asem[b]
            ).start()

    for i in (NI - 2, NI - 1):
        b = i % 2
        pltpu.make_async_copy(
            obuf[b], o_hbm.at[pl.ds(i * BI, BI), :], osem[b]
        ).wait()


@jax.jit
def kernel(input, adj, W):
    return pl.pallas_call(
        _fused_kernel,
        in_specs=[
            pl.BlockSpec(memory_space=pltpu.MemorySpace.HBM),
            pl.BlockSpec(memory_space=pltpu.MemorySpace.HBM),
            pl.BlockSpec(memory_space=pltpu.MemorySpace.HBM),
        ],
        out_specs=pl.BlockSpec(memory_space=pltpu.MemorySpace.HBM),
        out_shape=jax.ShapeDtypeStruct((N, D_OUT), jnp.float32),
        scratch_shapes=[
            pltpu.VMEM((N, D_IN), jnp.float32),
            pltpu.VMEM((D_IN, D_OUT), jnp.float32),
            pltpu.VMEM((N, D_OUT), jnp.bfloat16),
            pltpu.VMEM((BI, N), jnp.float32),
            pltpu.VMEM((BI, N), jnp.float32),
            pltpu.VMEM((BI, D_OUT), jnp.float32),
            pltpu.VMEM((BI, D_OUT), jnp.float32),
            pltpu.SemaphoreType.DMA,
            pltpu.SemaphoreType.DMA,
            pltpu.SemaphoreType.DMA,
            pltpu.SemaphoreType.DMA,
            pltpu.SemaphoreType.DMA,
            pltpu.SemaphoreType.DMA,
            pltpu.SemaphoreType.DMA,
        ],
    )(input, W, adj)


# speedup vs baseline: 1.0757x; 1.0757x over previous
import jax
import jax.numpy as jnp
from jax.experimental import pallas as pl
from jax.experimental.pallas import tpu as pltpu

N = 4096
D_IN = 512
D_OUT = 512
BI = 512
NI = N // BI


def _fused_kernel(x_hbm, w_hbm, adj_hbm, o_hbm,
                  x_v, w_v, h_v, a0, a1, o0, o1,
                  sx, sw, sa0, sa1, so0, so1):
    abuf = [a0, a1]
    asem = [sa0, sa1]
    obuf = [o0, o1]
    osem = [so0, so1]
    XH = N // 2

    cw = pltpu.make_async_copy(w_hbm, w_v, sw)
    cw.start()
    cx0 = pltpu.make_async_copy(
        x_hbm.at[pl.ds(0, XH), :], x_v.at[pl.ds(0, XH), :], sx
    )
    cx0.start()
    cw.wait()
    cx0.wait()
    cx1 = pltpu.make_async_copy(
        x_hbm.at[pl.ds(XH, XH), :], x_v.at[pl.ds(XH, XH), :], sx
    )
    cx1.start()
    h_v[pl.ds(0, XH), :] = jnp.dot(
        x_v[pl.ds(0, XH), :], w_v[...], preferred_element_type=jnp.float32
    ).astype(jnp.bfloat16)
    cx1.wait()
    for j in range(min(2, NI)):
        pltpu.make_async_copy(
            adj_hbm.at[pl.ds(j * BI, BI), :], abuf[j], asem[j]
        ).start()
    h_v[pl.ds(XH, XH), :] = jnp.dot(
        x_v[pl.ds(XH, XH), :], w_v[...], preferred_element_type=jnp.float32
    ).astype(jnp.bfloat16)

    for i in range(NI):
        b = i % 2
        pltpu.make_async_copy(
            adj_hbm.at[pl.ds(i * BI, BI), :], abuf[b], asem[b]
        ).wait()
        a = abuf[b][...]
        deg = jnp.sum(a, axis=1, keepdims=True)
        acc = jnp.dot(
            a.astype(jnp.bfloat16), h_v[...],
            preferred_element_type=jnp.float32,
        )
        if i >= 2:
            pltpu.make_async_copy(
                obuf[b], o_hbm.at[pl.ds((i - 2) * BI, BI), :], osem[b]
            ).wait()
        obuf[b][...] = acc / deg
        pltpu.make_async_copy(
            obuf[b], o_hbm.at[pl.ds(i * BI, BI), :], osem[b]
        ).start()
        if i + 2 < NI:
            pltpu.make_async_copy(
                adj_hbm.at[pl.ds((i + 2) * BI, BI), :], abuf[b], asem[b]
            ).start()

    for i in (NI - 2, NI - 1):
        b = i % 2
        pltpu.make_async_copy(
            obuf[b], o_hbm.at[pl.ds(i * BI, BI), :], osem[b]
        ).wait()


@jax.jit
def kernel(input, adj, W):
    return pl.pallas_call(
        _fused_kernel,
        in_specs=[
            pl.BlockSpec(memory_space=pltpu.MemorySpace.HBM),
            pl.BlockSpec(memory_space=pltpu.MemorySpace.HBM),
            pl.BlockSpec(memory_space=pltpu.MemorySpace.HBM),
        ],
        out_specs=pl.BlockSpec(memory_space=pltpu.MemorySpace.HBM),
        out_shape=jax.ShapeDtypeStruct((N, D_OUT), jnp.float32),
        scratch_shapes=[
            pltpu.VMEM((N, D_IN), jnp.float32),
            pltpu.VMEM((D_IN, D_OUT), jnp.float32),
            pltpu.VMEM((N, D_OUT), jnp.bfloat16),
            pltpu.VMEM((BI, N), jnp.float32),
            pltpu.VMEM((BI, N), jnp.float32),
            pltpu.VMEM((BI, D_OUT), jnp.float32),
            pltpu.VMEM((BI, D_OUT), jnp.float32),
            pltpu.SemaphoreType.DMA,
            pltpu.SemaphoreType.DMA,
            pltpu.SemaphoreType.DMA,
            pltpu.SemaphoreType.DMA,
            pltpu.SemaphoreType.DMA,
            pltpu.SemaphoreType.DMA,
        ],
    )(input, W, adj)
